# baseline (speedup 1.0000x reference)
import functools

import jax
import jax.numpy as jnp
from jax import lax
from jax.experimental import pallas as pl
from jax.experimental.pallas import tpu as pltpu
from jax.experimental.pallas import tpu_sc as plsc

_IDX = 13


@functools.lru_cache(maxsize=None)
def _make_sc_gather(V, D, B):
    info = plsc.get_sparse_core_info()
    NC, NS = info.num_cores, info.num_subcores
    NW = NC * NS
    b_per_w = B // NW
    LOOK = 16

    mesh = plsc.VectorSubcoreMesh(core_axis_name="c", subcore_axis_name="s")

    @functools.partial(
        pl.kernel,
        mesh=mesh,
        out_type=jax.ShapeDtypeStruct((B, D), jnp.float32),
        scratch_types=[
            pltpu.VMEM((b_per_w,), jnp.int32),
            pltpu.VMEM((b_per_w, D), jnp.float32),
            pltpu.SemaphoreType.DMA,
        ],
    )
    def gather_k(table_hbm, idx_hbm, out_hbm, idx_v, rows_v, sem):
        wid = lax.axis_index("s") * NC + lax.axis_index("c")
        base = wid * b_per_w
        n_ch = b_per_w // 16
        pltpu.sync_copy(idx_hbm.at[pl.ds(base, b_per_w)], idx_v)

        def fire_chunk(c):
            chunk = idx_v[pl.ds(c * 16, 16)]
            for k in range(16):
                s = chunk[k]
                pltpu.async_copy(
                    table_hbm.at[pl.ds(s, 1)],
                    rows_v.at[pl.ds(c * 16 + k, 1)],
                    sem,
                )

        def drain_chunk():
            for _ in range(16):
                pltpu.make_async_copy(
                    table_hbm.at[pl.ds(0, 1)], rows_v.at[pl.ds(0, 1)], sem
                ).wait()

        fire_chunk(0)

        def body(c, carry):
            fire_chunk(c + 1)
            drain_chunk()
            return carry

        lax.fori_loop(0, n_ch - 1, body, 0)
        drain_chunk()
        pltpu.sync_copy(rows_v, out_hbm.at[pl.ds(base, b_per_w)])

    return gather_k


def _mlp_body(x_ref, e_ref, w1d_ref, w1e_ref, b1_ref, w2_ref, b2_ref, o_ref):
    h = jnp.dot(x_ref[...], w1d_ref[...], preferred_element_type=jnp.float32)
    h = h + jnp.dot(e_ref[...], w1e_ref[...], preferred_element_type=jnp.float32)
    h = jnp.maximum(h + b1_ref[...], 0.0)
    o = jnp.dot(h, w2_ref[...], preferred_element_type=jnp.float32) + b2_ref[...]
    o_ref[...] = jnp.maximum(o, 0.0)


def _mlp(x, emb, W1d, W1e, b1, W2, b2, block_b=2048):
    B, F = x.shape
    HID = W2.shape[0]
    OUT = W2.shape[1]
    D = emb.shape[1]
    grid = (B // block_b,)
    return pl.pallas_call(
        _mlp_body,
        grid=grid,
        in_specs=[
            pl.BlockSpec((block_b, F), lambda i: (i, 0)),
            pl.BlockSpec((block_b, D), lambda i: (i, 0)),
            pl.BlockSpec((F, HID), lambda i: (0, 0)),
            pl.BlockSpec((D, HID), lambda i: (0, 0)),
            pl.BlockSpec((1, HID), lambda i: (0, 0)),
            pl.BlockSpec((HID, OUT), lambda i: (0, 0)),
            pl.BlockSpec((1, OUT), lambda i: (0, 0)),
        ],
        out_specs=pl.BlockSpec((block_b, OUT), lambda i: (i, 0)),
        out_shape=jax.ShapeDtypeStruct((B, OUT), jnp.float32),
    )(x, emb, W1d, W1e, b1, W2, b2)


def kernel(inputs, table, W1, b1, W2, b2):
    B, F = inputs.shape
    V, D = table.shape
    HID = W1.shape[1]
    idx = inputs[:, _IDX].astype(jnp.int32)
    emb = _make_sc_gather(V, D, B)(table, idx)
    W1d = jnp.concatenate(
        [W1[:_IDX], jnp.zeros((1, HID), W1.dtype), W1[_IDX : F - 1]], axis=0
    )
    W1e = W1[F - 1 :]
    return _mlp(inputs, emb, W1d, W1e, b1.reshape(1, -1), W2, b2.reshape(1, -1))
